# baseline (speedup 1.0000x reference)
import jax
import jax.numpy as jnp
from jax.experimental import pallas as pl

B = 64
M = 1000
T = 30
XLEN = 128
K = 6
H = 64
ALPHA = 0.01
THRESH = 0.1
NEG = -1e30


def _lane_mean(x, n):
    s = x * (1.0 / n)
    g = s[:, 0:8]
    for i in range(1, n // 8):
        g = g + s[:, 8 * i:8 * i + 8]
    w = 8
    while w > 1:
        w //= 2
        g = g[:, :w] + g[:, w:2 * w]
    return jax.lax.broadcast_in_dim(g, x.shape, (0, 1))


def _ln(x, g, b):
    mu = _lane_mean(x, x.shape[-1])
    var = _lane_mean((x - mu) ** 2, x.shape[-1])
    return (x - mu) / jnp.sqrt(var + 1e-5) * g + b


def _erfc(x):
    f = jnp.float32
    ax = jnp.abs(x)
    x2 = x * x
    p = x2 * f(7.85386146e-05) + f(-0.000801019371)
    p = p * x2 + f(0.00518832775)
    p = p * x2 + f(-0.0268538129)
    p = p * x2 + f(0.112835854)
    p = p * x2 + f(-0.37612626)
    p = p * x2 + f(1.12837911)
    one_minus_erf = f(1.0) - x * p

    z = -x2
    ez = jnp.exp(z)
    q = ez * (f(1.0) / ax)
    w = f(1.0) / x2
    a = w * f(0.0232682) + f(-0.138703942)
    a = a * w + f(0.368742466)
    a = a * w + f(-0.582473278)
    a = a * w + f(0.621000469)
    a = a * w + f(-0.494451523)
    a = a * w + f(0.340488)
    a = a * w + f(-0.274112701)
    a = a * w + f(0.563825965)
    bb = w * f(-10.477664) + f(12.9772)
    bb = bb * w + f(-7.49551868)
    bb = bb * w + f(2.92101908)
    bb = bb * w + f(-1.01526523)
    bb = bb * w + f(0.42184633)
    bb = bb * w + f(-0.282076746)
    bb = bb * w + f(0.564189494)
    y = q * jnp.where(ax < f(2.0), a, bb)
    y = jnp.where(z < f(-88.7228394), f(0.0), y)
    y = jnp.where(x < f(0.0), f(2.0) - y, y)
    return jnp.where(ax < f(1.0), one_minus_erf, y)


def _gelu(x):
    return 0.5 * x * _erfc(-x / jnp.sqrt(2.0).astype(jnp.float32))


def _sortkey(x):
    b = jax.lax.bitcast_convert_type(x, jnp.int32)
    return jnp.where(b < 0, b ^ jnp.int32(0x7FFFFFFF), b)


def _planes(ints):
    out = []
    for i in range(4):
        c = jax.lax.shift_right_logical(ints, jnp.int32(8 * i))
        out.append(jnp.bitwise_and(c, jnp.int32(255)).astype(jnp.float32))
    return out


def _unplanes(planes):
    acc = jnp.zeros(planes[0].shape, jnp.int32)
    for i in range(4):
        c = planes[i].astype(jnp.int32)
        acc = jnp.bitwise_or(acc, jax.lax.shift_left(c, jnp.int32(8 * i)))
    return acc


def _exact_rowmix(onehot, mat):
    bits = jax.lax.bitcast_convert_type(mat, jnp.int32)
    mixed = [jnp.dot(onehot, p, preferred_element_type=jnp.float32)
             for p in _planes(bits)]
    return jax.lax.bitcast_convert_type(_unplanes(mixed), jnp.float32)


def _body(pre_ref, px_ref, py_ref, at_ref, vf_ref, lx_ref, ly_ref,
          w1p_ref, w1a_ref, w1v_ref, b1_ref, g1_ref, be1_ref,
          w2_ref, b2_ref, g2_ref, be2_ref, w3_ref, b3_ref,
          eye_ref, ltri_ref, tie_ref,
          l3_ref, traj_ref, dis_ref):
    b = pl.program_id(0)

    pre = pre_ref[0]
    px = px_ref[0]
    py = py_ref[0]
    vfb = jnp.broadcast_to(vf_ref[0], (M, XLEN))
    atb = jnp.broadcast_to(at_ref[0], (M, XLEN))
    feat = jnp.concatenate([vfb, atb, pre], axis=1)
    w1full = jnp.concatenate([w1v_ref[...], w1a_ref[...], w1p_ref[...]], axis=0)
    h = jnp.dot(feat, w1full, preferred_element_type=jnp.float32) + b1_ref[...]
    h = _gelu(_ln(h, g1_ref[...], be1_ref[...]))
    h = jnp.dot(h, w2_ref[...], preferred_element_type=jnp.float32) + b2_ref[...]
    h = _gelu(_ln(h, g2_ref[...], be2_ref[...]))
    pd = jnp.dot(h, w3_ref[...], preferred_element_type=jnp.float32) + b3_ref[...]
    pd = pd - jnp.max(pd, axis=0, keepdims=True)
    pd = pd - jnp.log(jnp.sum(jnp.exp(pd), axis=0, keepdims=True))

    d2t = (px - lx_ref[0]) ** 2 + (py - ly_ref[0]) ** 2
    z = -jnp.max(d2t, axis=1, keepdims=True) / ALPHA
    z = z - jnp.max(z, axis=0, keepdims=True)
    e = jnp.exp(z)
    dis = e / jnp.sum(e, axis=0, keepdims=True)
    pos = dis > 0
    logdis = jnp.log(jnp.where(pos, dis, 1.0))
    l3c = jnp.sum(jnp.where(pos, dis * (logdis - pd), 0.0),
                  keepdims=True) * (1.0 / B)

    @pl.when(b == 0)
    def _():
        l3_ref[...] = jnp.zeros((1, 1), jnp.float32)
    l3_ref[...] += l3c

    iota_c = jax.lax.broadcasted_iota(jnp.int32, (M, 1), 0)
    eye = eye_ref[...]
    ltri = ltri_ref[...]

    key = _sortkey(pd)
    keyrow_p = [jax.lax.dot_general(p, eye, (((0,), (0,)), ((), ())),
                                    preferred_element_type=jnp.float32)
                for p in _planes(key)]
    keyrow = _unplanes(keyrow_p)
    g = jnp.where((key > keyrow) | ((key == keyrow) & tie_ref[...]), 1.0, 0.0)
    rank_row = jnp.dot(jnp.ones((1, M), jnp.float32), g,
                       preferred_element_type=jnp.float32)
    perm = jnp.where(iota_c.astype(jnp.float32) == rank_row, 1.0, 0.0)

    n2 = px[:, 0:1] ** 2 + py[:, 0:1] ** 2
    mask5c = jnp.where(jnp.sqrt(n2) > 5.0, 1.0, 0.0)
    mask5p = jnp.dot(perm, mask5c, preferred_element_type=jnp.float32) > 0.5

    v = pd
    avail = jnp.ones((M, 1), dtype=jnp.bool_)
    accum = jnp.zeros((M, T), dtype=jnp.float32)
    iota_k = jax.lax.broadcasted_iota(jnp.int32, (1, K), 1)
    dvec = jnp.zeros((1, K), jnp.float32)
    for k in range(K):
        if k == 0:
            pick = iota_c == 0
        else:
            eff = jnp.where(avail, v, NEG)
            mx = jnp.max(eff, axis=0, keepdims=True)
            hit = eff == mx
            cs = jnp.dot(ltri, jnp.where(hit, 1.0, 0.0),
                         preferred_element_type=jnp.float32)
            pick = hit & (cs == 1.0)
        pick_f = jnp.where(pick, 1.0, 0.0)
        vsel = jnp.sum(jnp.where(pick, v, 0.0))
        dvec = jnp.where(iota_k == k, jnp.exp(vsel), dvec)
        onehot_c = jax.lax.dot_general(pick_f, perm, (((0,), (0,)), ((), ())),
                                       preferred_element_type=jnp.float32)
        row = _exact_rowmix(onehot_c, pre)
        rowx = _exact_rowmix(onehot_c, px)
        rowy = _exact_rowmix(onehot_c, py)
        traj_ref[0, pl.ds(k, 1), :] = row
        avail = avail & jnp.logical_not(pick)
        v = jnp.where(mask5p, -1.0, v)
        d2 = (px - rowx) ** 2 + (py - rowy) ** 2
        accum = accum + jnp.sqrt(d2)
        m2c = jnp.where(
            jnp.max(accum, axis=1, keepdims=True) * (1.0 / (k + 1)) < THRESH,
            1.0, 0.0)
        m2p = jnp.dot(perm, m2c, preferred_element_type=jnp.float32) > 0.5
        v = jnp.where(m2p, 1.1 * v, v)
    dis_ref[0] = dvec


def kernel(agent_traj, pre, labels, vectornet_feature,
           W1, b1, g1, be1, W2, b2, g2, be2, W3, b3):
    w1v = W1[:XLEN]
    w1a = W1[XLEN:2 * XLEN]
    w1p = W1[2 * XLEN:]
    pre_r = pre.reshape(B, M, T, 2)
    px = pre_r[..., 0]
    py = pre_r[..., 1]
    labx = labels[..., 0].reshape(B, 1, T)
    laby = labels[..., 1].reshape(B, 1, T)

    r = jnp.arange(M)
    eye_c = (r[:, None] == r[None, :]).astype(jnp.float32)
    ltri_c = (r[None, :] <= r[:, None]).astype(jnp.float32)
    tie_c = r[:, None] < r[None, :]

    fixed = lambda *_: (0, 0)
    l3, traj, dislist = pl.pallas_call(
        _body,
        grid=(B,),
        in_specs=[
            pl.BlockSpec((1, M, 2 * T), lambda b: (b, 0, 0)),
            pl.BlockSpec((1, M, T), lambda b: (b, 0, 0)),
            pl.BlockSpec((1, M, T), lambda b: (b, 0, 0)),
            pl.BlockSpec((1, 1, XLEN), lambda b: (b, 0, 0)),
            pl.BlockSpec((1, 1, XLEN), lambda b: (b, 0, 0)),
            pl.BlockSpec((1, 1, T), lambda b: (b, 0, 0)),
            pl.BlockSpec((1, 1, T), lambda b: (b, 0, 0)),
            pl.BlockSpec((2 * T, H), fixed),
            pl.BlockSpec((XLEN, H), fixed),
            pl.BlockSpec((XLEN, H), fixed),
            pl.BlockSpec((1, H), fixed),
            pl.BlockSpec((1, H), fixed),
            pl.BlockSpec((1, H), fixed),
            pl.BlockSpec((H, H), fixed),
            pl.BlockSpec((1, H), fixed),
            pl.BlockSpec((1, H), fixed),
            pl.BlockSpec((1, H), fixed),
            pl.BlockSpec((H, 1), fixed),
            pl.BlockSpec((1, 1), fixed),
            pl.BlockSpec((M, M), fixed),
            pl.BlockSpec((M, M), fixed),
            pl.BlockSpec((M, M), fixed),
        ],
        out_specs=[
            pl.BlockSpec((1, 1), fixed),
            pl.BlockSpec((1, K, 2 * T), lambda b: (b, 0, 0)),
            pl.BlockSpec((1, 1, K), lambda b: (b, 0, 0)),
        ],
        out_shape=[
            jax.ShapeDtypeStruct((1, 1), jnp.float32),
            jax.ShapeDtypeStruct((B, K, 2 * T), jnp.float32),
            jax.ShapeDtypeStruct((B, 1, K), jnp.float32),
        ],
    )(pre, px, py,
      agent_traj.reshape(B, 1, XLEN), vectornet_feature.reshape(B, 1, XLEN),
      labx, laby,
      w1p, w1a, w1v, b1.reshape(1, H), g1.reshape(1, H), be1.reshape(1, H),
      W2, b2.reshape(1, H), g2.reshape(1, H), be2.reshape(1, H),
      W3, b3.reshape(1, 1), eye_c, ltri_c, tie_c)
    return l3[0, 0], traj.reshape(B, K, T, 2), dislist.reshape(B, K)


# speedup vs baseline: 1.0462x; 1.0462x over previous
import jax
import jax.numpy as jnp
from jax.experimental import pallas as pl

B = 64
M = 1000
T = 30
XLEN = 128
K = 6
H = 64
ALPHA = 0.01
THRESH = 0.1
NEG = -1e30


def _lane_mean(x, n):
    s = x * (1.0 / n)
    g = s[:, 0:8]
    for i in range(1, n // 8):
        g = g + s[:, 8 * i:8 * i + 8]
    w = 8
    while w > 1:
        w //= 2
        g = g[:, :w] + g[:, w:2 * w]
    return jax.lax.broadcast_in_dim(g, x.shape, (0, 1))


def _ln(x, g, b):
    mu = _lane_mean(x, x.shape[-1])
    var = _lane_mean((x - mu) ** 2, x.shape[-1])
    return (x - mu) / jnp.sqrt(var + 1e-5) * g + b


def _erfc(x):
    f = jnp.float32
    ax = jnp.abs(x)
    x2 = x * x
    p = x2 * f(7.85386146e-05) + f(-0.000801019371)
    p = p * x2 + f(0.00518832775)
    p = p * x2 + f(-0.0268538129)
    p = p * x2 + f(0.112835854)
    p = p * x2 + f(-0.37612626)
    p = p * x2 + f(1.12837911)
    one_minus_erf = f(1.0) - x * p

    z = -x2
    ez = jnp.exp(z)
    q = ez * (f(1.0) / ax)
    w = f(1.0) / x2
    a = w * f(0.0232682) + f(-0.138703942)
    a = a * w + f(0.368742466)
    a = a * w + f(-0.582473278)
    a = a * w + f(0.621000469)
    a = a * w + f(-0.494451523)
    a = a * w + f(0.340488)
    a = a * w + f(-0.274112701)
    a = a * w + f(0.563825965)
    bb = w * f(-10.477664) + f(12.9772)
    bb = bb * w + f(-7.49551868)
    bb = bb * w + f(2.92101908)
    bb = bb * w + f(-1.01526523)
    bb = bb * w + f(0.42184633)
    bb = bb * w + f(-0.282076746)
    bb = bb * w + f(0.564189494)
    y = q * jnp.where(ax < f(2.0), a, bb)
    y = jnp.where(z < f(-88.7228394), f(0.0), y)
    y = jnp.where(x < f(0.0), f(2.0) - y, y)
    return jnp.where(ax < f(1.0), one_minus_erf, y)


def _gelu(x):
    return 0.5 * x * _erfc(-x / jnp.sqrt(2.0).astype(jnp.float32))


def _sortkey(x):
    b = jax.lax.bitcast_convert_type(x, jnp.int32)
    return jnp.where(b < 0, b ^ jnp.int32(0x7FFFFFFF), b)


def _planes(ints):
    out = []
    for i in range(4):
        c = jax.lax.shift_right_logical(ints, jnp.int32(8 * i))
        out.append(jnp.bitwise_and(c, jnp.int32(255)).astype(jnp.float32))
    return out


def _unplanes(planes):
    acc = jnp.zeros(planes[0].shape, jnp.int32)
    for i in range(4):
        c = planes[i].astype(jnp.int32)
        acc = jnp.bitwise_or(acc, jax.lax.shift_left(c, jnp.int32(8 * i)))
    return acc


def _exact_rowmix(onehot, mat):
    bits = jax.lax.bitcast_convert_type(mat, jnp.int32)
    mixed = [jnp.dot(onehot, p, preferred_element_type=jnp.float32)
             for p in _planes(bits)]
    return jax.lax.bitcast_convert_type(_unplanes(mixed), jnp.float32)


def _body(pre_ref, px_ref, py_ref, at_ref, vf_ref, lx_ref, ly_ref,
          w1p_ref, w1a_ref, w1v_ref, b1_ref, g1_ref, be1_ref,
          w2_ref, b2_ref, g2_ref, be2_ref, w3_ref, b3_ref,
          l3_ref, traj_ref, dis_ref):
    b = pl.program_id(0)

    pre = pre_ref[0]
    px = px_ref[0]
    py = py_ref[0]
    vfb = jnp.broadcast_to(vf_ref[0], (M, XLEN))
    atb = jnp.broadcast_to(at_ref[0], (M, XLEN))
    feat = jnp.concatenate([vfb, atb, pre], axis=1)
    w1full = jnp.concatenate([w1v_ref[...], w1a_ref[...], w1p_ref[...]], axis=0)
    h = jnp.dot(feat, w1full, preferred_element_type=jnp.float32) + b1_ref[...]
    h = _gelu(_ln(h, g1_ref[...], be1_ref[...]))
    h = jnp.dot(h, w2_ref[...], preferred_element_type=jnp.float32) + b2_ref[...]
    h = _gelu(_ln(h, g2_ref[...], be2_ref[...]))
    pd = jnp.dot(h, w3_ref[...], preferred_element_type=jnp.float32) + b3_ref[...]
    pd = pd - jnp.max(pd, axis=0, keepdims=True)
    pd = pd - jnp.log(jnp.sum(jnp.exp(pd), axis=0, keepdims=True))

    d2t = (px - lx_ref[0]) ** 2 + (py - ly_ref[0]) ** 2
    z = -jnp.max(d2t, axis=1, keepdims=True) / ALPHA
    z = z - jnp.max(z, axis=0, keepdims=True)
    e = jnp.exp(z)
    dis = e / jnp.sum(e, axis=0, keepdims=True)
    pos = dis > 0
    logdis = jnp.log(jnp.where(pos, dis, 1.0))
    l3c = jnp.sum(jnp.where(pos, dis * (logdis - pd), 0.0),
                  keepdims=True) * (1.0 / B)

    @pl.when(b == 0)
    def _():
        l3_ref[...] = jnp.zeros((1, 1), jnp.float32)
    l3_ref[...] += l3c

    iota_c = jax.lax.broadcasted_iota(jnp.int32, (M, 1), 0)
    rr2 = jax.lax.broadcasted_iota(jnp.int32, (M, M), 0)
    cc2 = jax.lax.broadcasted_iota(jnp.int32, (M, M), 1)
    eye = jnp.where(rr2 == cc2, 1.0, 0.0)
    ltri = jnp.where(cc2 <= rr2, 1.0, 0.0)

    key = _sortkey(pd)
    keyrow_p = [jax.lax.dot_general(p, eye, (((0,), (0,)), ((), ())),
                                    preferred_element_type=jnp.float32)
                for p in _planes(key)]
    keyrow = _unplanes(keyrow_p)
    g = jnp.where((key > keyrow) | ((key == keyrow) & (rr2 < cc2)), 1.0, 0.0)
    rank_row = jnp.dot(jnp.ones((1, M), jnp.float32), g,
                       preferred_element_type=jnp.float32)
    perm = jnp.where(iota_c.astype(jnp.float32) == rank_row, 1.0, 0.0)

    n2 = px[:, 0:1] ** 2 + py[:, 0:1] ** 2
    mask5c = jnp.where(jnp.sqrt(n2) > 5.0, 1.0, 0.0)
    mask5p = jnp.dot(perm, mask5c, preferred_element_type=jnp.float32) > 0.5

    v = pd
    avail = jnp.ones((M, 1), dtype=jnp.bool_)
    accum = jnp.zeros((M, T), dtype=jnp.float32)
    iota_k = jax.lax.broadcasted_iota(jnp.int32, (1, K), 1)
    dvec = jnp.zeros((1, K), jnp.float32)
    for k in range(K):
        if k == 0:
            pick = iota_c == 0
        else:
            eff = jnp.where(avail, v, NEG)
            mx = jnp.max(eff, axis=0, keepdims=True)
            hit = eff == mx
            cs = jnp.dot(ltri, jnp.where(hit, 1.0, 0.0),
                         preferred_element_type=jnp.float32)
            pick = hit & (cs == 1.0)
        pick_f = jnp.where(pick, 1.0, 0.0)
        vsel = jnp.sum(jnp.where(pick, v, 0.0))
        dvec = jnp.where(iota_k == k, jnp.exp(vsel), dvec)
        onehot_c = jax.lax.dot_general(pick_f, perm, (((0,), (0,)), ((), ())),
                                       preferred_element_type=jnp.float32)
        row = _exact_rowmix(onehot_c, pre)
        rowx = _exact_rowmix(onehot_c, px)
        rowy = _exact_rowmix(onehot_c, py)
        traj_ref[0, pl.ds(k, 1), :] = row
        avail = avail & jnp.logical_not(pick)
        v = jnp.where(mask5p, -1.0, v)
        d2 = (px - rowx) ** 2 + (py - rowy) ** 2
        accum = accum + jnp.sqrt(d2)
        m2c = jnp.where(
            jnp.max(accum, axis=1, keepdims=True) * (1.0 / (k + 1)) < THRESH,
            1.0, 0.0)
        m2p = jnp.dot(perm, m2c, preferred_element_type=jnp.float32) > 0.5
        v = jnp.where(m2p, 1.1 * v, v)
    dis_ref[0] = dvec


def kernel(agent_traj, pre, labels, vectornet_feature,
           W1, b1, g1, be1, W2, b2, g2, be2, W3, b3):
    w1v = W1[:XLEN]
    w1a = W1[XLEN:2 * XLEN]
    w1p = W1[2 * XLEN:]
    pre_r = pre.reshape(B, M, T, 2)
    px = pre_r[..., 0]
    py = pre_r[..., 1]
    labx = labels[..., 0].reshape(B, 1, T)
    laby = labels[..., 1].reshape(B, 1, T)

    fixed = lambda *_: (0, 0)
    l3, traj, dislist = pl.pallas_call(
        _body,
        grid=(B,),
        in_specs=[
            pl.BlockSpec((1, M, 2 * T), lambda b: (b, 0, 0)),
            pl.BlockSpec((1, M, T), lambda b: (b, 0, 0)),
            pl.BlockSpec((1, M, T), lambda b: (b, 0, 0)),
            pl.BlockSpec((1, 1, XLEN), lambda b: (b, 0, 0)),
            pl.BlockSpec((1, 1, XLEN), lambda b: (b, 0, 0)),
            pl.BlockSpec((1, 1, T), lambda b: (b, 0, 0)),
            pl.BlockSpec((1, 1, T), lambda b: (b, 0, 0)),
            pl.BlockSpec((2 * T, H), fixed),
            pl.BlockSpec((XLEN, H), fixed),
            pl.BlockSpec((XLEN, H), fixed),
            pl.BlockSpec((1, H), fixed),
            pl.BlockSpec((1, H), fixed),
            pl.BlockSpec((1, H), fixed),
            pl.BlockSpec((H, H), fixed),
            pl.BlockSpec((1, H), fixed),
            pl.BlockSpec((1, H), fixed),
            pl.BlockSpec((1, H), fixed),
            pl.BlockSpec((H, 1), fixed),
            pl.BlockSpec((1, 1), fixed),
        ],
        out_specs=[
            pl.BlockSpec((1, 1), fixed),
            pl.BlockSpec((1, K, 2 * T), lambda b: (b, 0, 0)),
            pl.BlockSpec((1, 1, K), lambda b: (b, 0, 0)),
        ],
        out_shape=[
            jax.ShapeDtypeStruct((1, 1), jnp.float32),
            jax.ShapeDtypeStruct((B, K, 2 * T), jnp.float32),
            jax.ShapeDtypeStruct((B, 1, K), jnp.float32),
        ],
    )(pre, px, py,
      agent_traj.reshape(B, 1, XLEN), vectornet_feature.reshape(B, 1, XLEN),
      labx, laby,
      w1p, w1a, w1v, b1.reshape(1, H), g1.reshape(1, H), be1.reshape(1, H),
      W2, b2.reshape(1, H), g2.reshape(1, H), be2.reshape(1, H),
      W3, b3.reshape(1, 1))
    return l3[0, 0], traj.reshape(B, K, T, 2), dislist.reshape(B, K)


# speedup vs baseline: 1.0809x; 1.0332x over previous
import jax
import jax.numpy as jnp
from jax.experimental import pallas as pl

B = 64
M = 1000
T = 30
XLEN = 128
K = 6
H = 64
ALPHA = 0.01
THRESH = 0.1
NEG = -1e30


def _lane_mean(x, n):
    s = x * (1.0 / n)
    g = s[:, 0:8]
    for i in range(1, n // 8):
        g = g + s[:, 8 * i:8 * i + 8]
    w = 8
    while w > 1:
        w //= 2
        g = g[:, :w] + g[:, w:2 * w]
    return jax.lax.broadcast_in_dim(g, x.shape, (0, 1))


def _ln(x, g, b):
    mu = _lane_mean(x, x.shape[-1])
    var = _lane_mean((x - mu) ** 2, x.shape[-1])
    return (x - mu) / jnp.sqrt(var + 1e-5) * g + b


def _erfc(x):
    f = jnp.float32
    ax = jnp.abs(x)
    x2 = x * x
    p = x2 * f(7.85386146e-05) + f(-0.000801019371)
    p = p * x2 + f(0.00518832775)
    p = p * x2 + f(-0.0268538129)
    p = p * x2 + f(0.112835854)
    p = p * x2 + f(-0.37612626)
    p = p * x2 + f(1.12837911)
    one_minus_erf = f(1.0) - x * p

    z = -x2
    ez = jnp.exp(z)
    q = ez * (f(1.0) / ax)
    w = f(1.0) / x2
    a = w * f(0.0232682) + f(-0.138703942)
    a = a * w + f(0.368742466)
    a = a * w + f(-0.582473278)
    a = a * w + f(0.621000469)
    a = a * w + f(-0.494451523)
    a = a * w + f(0.340488)
    a = a * w + f(-0.274112701)
    a = a * w + f(0.563825965)
    bb = w * f(-10.477664) + f(12.9772)
    bb = bb * w + f(-7.49551868)
    bb = bb * w + f(2.92101908)
    bb = bb * w + f(-1.01526523)
    bb = bb * w + f(0.42184633)
    bb = bb * w + f(-0.282076746)
    bb = bb * w + f(0.564189494)
    y = q * jnp.where(ax < f(2.0), a, bb)
    y = jnp.where(z < f(-88.7228394), f(0.0), y)
    y = jnp.where(x < f(0.0), f(2.0) - y, y)
    return jnp.where(ax < f(1.0), one_minus_erf, y)


def _gelu(x):
    return 0.5 * x * _erfc(-x / jnp.sqrt(2.0).astype(jnp.float32))


def _sortkey(x):
    b = jax.lax.bitcast_convert_type(x, jnp.int32)
    return jnp.where(b < 0, b ^ jnp.int32(0x7FFFFFFF), b)


def _planes(ints):
    out = []
    for i in range(4):
        c = jax.lax.shift_right_logical(ints, jnp.int32(8 * i))
        out.append(jnp.bitwise_and(c, jnp.int32(255)).astype(jnp.float32))
    return out


def _unplanes(planes):
    acc = jnp.zeros(planes[0].shape, jnp.int32)
    for i in range(4):
        c = planes[i].astype(jnp.int32)
        acc = jnp.bitwise_or(acc, jax.lax.shift_left(c, jnp.int32(8 * i)))
    return acc


def _exact_rowmix(onehot, planes):
    mixed = [jnp.dot(onehot, p, preferred_element_type=jnp.float32)
             for p in planes]
    return jax.lax.bitcast_convert_type(_unplanes(mixed), jnp.float32)


def _body(pre_ref, px_ref, py_ref, at_ref, vf_ref, lx_ref, ly_ref,
          w1p_ref, w1a_ref, w1v_ref, b1_ref, g1_ref, be1_ref,
          w2_ref, b2_ref, g2_ref, be2_ref, w3_ref, b3_ref,
          l3_ref, traj_ref, dis_ref):
    b = pl.program_id(0)

    pre = pre_ref[0]
    px = px_ref[0]
    py = py_ref[0]
    vfb = jnp.broadcast_to(vf_ref[0], (M, XLEN))
    atb = jnp.broadcast_to(at_ref[0], (M, XLEN))
    feat = jnp.concatenate([vfb, atb, pre], axis=1)
    w1full = jnp.concatenate([w1v_ref[...], w1a_ref[...], w1p_ref[...]], axis=0)
    h = jnp.dot(feat, w1full, preferred_element_type=jnp.float32) + b1_ref[...]
    h = _gelu(_ln(h, g1_ref[...], be1_ref[...]))
    h = jnp.dot(h, w2_ref[...], preferred_element_type=jnp.float32) + b2_ref[...]
    h = _gelu(_ln(h, g2_ref[...], be2_ref[...]))
    pd = jnp.dot(h, w3_ref[...], preferred_element_type=jnp.float32) + b3_ref[...]
    pd = pd - jnp.max(pd, axis=0, keepdims=True)
    pd = pd - jnp.log(jnp.sum(jnp.exp(pd), axis=0, keepdims=True))

    d2t = (px - lx_ref[0]) ** 2 + (py - ly_ref[0]) ** 2
    z = -jnp.max(d2t, axis=1, keepdims=True) / ALPHA
    z = z - jnp.max(z, axis=0, keepdims=True)
    e = jnp.exp(z)
    dis = e / jnp.sum(e, axis=0, keepdims=True)
    pos = dis > 0
    logdis = jnp.log(jnp.where(pos, dis, 1.0))
    l3c = jnp.sum(jnp.where(pos, dis * (logdis - pd), 0.0),
                  keepdims=True) * (1.0 / B)

    @pl.when(b == 0)
    def _():
        l3_ref[...] = jnp.zeros((1, 1), jnp.float32)
    l3_ref[...] += l3c

    iota_c = jax.lax.broadcasted_iota(jnp.int32, (M, 1), 0)
    rr2 = jax.lax.broadcasted_iota(jnp.int32, (M, M), 0)
    cc2 = jax.lax.broadcasted_iota(jnp.int32, (M, M), 1)
    eye = jnp.where(rr2 == cc2, 1.0, 0.0)
    ltri = jnp.where(cc2 <= rr2, 1.0, 0.0)

    key = _sortkey(pd)
    pl4 = jnp.concatenate(_planes(key), axis=1)
    tr4 = jax.lax.dot_general(pl4, eye, (((0,), (0,)), ((), ())),
                              preferred_element_type=jnp.float32)
    keyrow = _unplanes([tr4[i:i + 1] for i in range(4)])
    g = jnp.where((key > keyrow) | ((key == keyrow) & (rr2 < cc2)), 1.0, 0.0)
    rank_row = jnp.dot(jnp.ones((1, M), jnp.float32), g,
                       preferred_element_type=jnp.float32)
    perm = jnp.where(iota_c.astype(jnp.float32) == rank_row, 1.0, 0.0)

    n2 = px[:, 0:1] ** 2 + py[:, 0:1] ** 2
    mask5c = jnp.where(jnp.sqrt(n2) > 5.0, 1.0, 0.0)
    mask5p = jnp.dot(perm, mask5c, preferred_element_type=jnp.float32) > 0.5

    hcat = jnp.concatenate([pre, px, py], axis=1)
    hplanes = _planes(jax.lax.bitcast_convert_type(hcat, jnp.int32))

    v = pd
    avail = jnp.ones((M, 1), dtype=jnp.bool_)
    accum = jnp.zeros((M, T), dtype=jnp.float32)
    iota_k = jax.lax.broadcasted_iota(jnp.int32, (1, K), 1)
    dvec = jnp.zeros((1, K), jnp.float32)
    for k in range(K):
        if k == 0:
            pick = iota_c == 0
        else:
            eff = jnp.where(avail, v, NEG)
            mx = jnp.max(eff, axis=0, keepdims=True)
            hit = eff == mx
            cs = jnp.dot(ltri, jnp.where(hit, 1.0, 0.0),
                         preferred_element_type=jnp.float32)
            pick = hit & (cs == 1.0)
        pick_f = jnp.where(pick, 1.0, 0.0)
        vsel = jnp.sum(jnp.where(pick, v, 0.0))
        dvec = jnp.where(iota_k == k, jnp.exp(vsel), dvec)
        onehot_c = jax.lax.dot_general(pick_f, perm, (((0,), (0,)), ((), ())),
                                       preferred_element_type=jnp.float32)
        rcat = _exact_rowmix(onehot_c, hplanes)
        row = rcat[:, :2 * T]
        rowx = rcat[:, 2 * T:3 * T]
        rowy = rcat[:, 3 * T:]
        traj_ref[0, pl.ds(k, 1), :] = row
        avail = avail & jnp.logical_not(pick)
        v = jnp.where(mask5p, -1.0, v)
        d2 = (px - rowx) ** 2 + (py - rowy) ** 2
        accum = accum + jnp.sqrt(d2)
        m2c = jnp.where(
            jnp.max(accum, axis=1, keepdims=True) * (1.0 / (k + 1)) < THRESH,
            1.0, 0.0)
        m2p = jnp.dot(perm, m2c, preferred_element_type=jnp.float32) > 0.5
        v = jnp.where(m2p, 1.1 * v, v)
    dis_ref[0] = dvec


def kernel(agent_traj, pre, labels, vectornet_feature,
           W1, b1, g1, be1, W2, b2, g2, be2, W3, b3):
    w1v = W1[:XLEN]
    w1a = W1[XLEN:2 * XLEN]
    w1p = W1[2 * XLEN:]
    pre_r = pre.reshape(B, M, T, 2)
    px = pre_r[..., 0]
    py = pre_r[..., 1]
    labx = labels[..., 0].reshape(B, 1, T)
    laby = labels[..., 1].reshape(B, 1, T)

    fixed = lambda *_: (0, 0)
    l3, traj, dislist = pl.pallas_call(
        _body,
        grid=(B,),
        in_specs=[
            pl.BlockSpec((1, M, 2 * T), lambda b: (b, 0, 0)),
            pl.BlockSpec((1, M, T), lambda b: (b, 0, 0)),
            pl.BlockSpec((1, M, T), lambda b: (b, 0, 0)),
            pl.BlockSpec((1, 1, XLEN), lambda b: (b, 0, 0)),
            pl.BlockSpec((1, 1, XLEN), lambda b: (b, 0, 0)),
            pl.BlockSpec((1, 1, T), lambda b: (b, 0, 0)),
            pl.BlockSpec((1, 1, T), lambda b: (b, 0, 0)),
            pl.BlockSpec((2 * T, H), fixed),
            pl.BlockSpec((XLEN, H), fixed),
            pl.BlockSpec((XLEN, H), fixed),
            pl.BlockSpec((1, H), fixed),
            pl.BlockSpec((1, H), fixed),
            pl.BlockSpec((1, H), fixed),
            pl.BlockSpec((H, H), fixed),
            pl.BlockSpec((1, H), fixed),
            pl.BlockSpec((1, H), fixed),
            pl.BlockSpec((1, H), fixed),
            pl.BlockSpec((H, 1), fixed),
            pl.BlockSpec((1, 1), fixed),
        ],
        out_specs=[
            pl.BlockSpec((1, 1), fixed),
            pl.BlockSpec((1, K, 2 * T), lambda b: (b, 0, 0)),
            pl.BlockSpec((1, 1, K), lambda b: (b, 0, 0)),
        ],
        out_shape=[
            jax.ShapeDtypeStruct((1, 1), jnp.float32),
            jax.ShapeDtypeStruct((B, K, 2 * T), jnp.float32),
            jax.ShapeDtypeStruct((B, 1, K), jnp.float32),
        ],
    )(pre, px, py,
      agent_traj.reshape(B, 1, XLEN), vectornet_feature.reshape(B, 1, XLEN),
      labx, laby,
      w1p, w1a, w1v, b1.reshape(1, H), g1.reshape(1, H), be1.reshape(1, H),
      W2, b2.reshape(1, H), g2.reshape(1, H), be2.reshape(1, H),
      W3, b3.reshape(1, 1))
    return l3[0, 0], traj.reshape(B, K, T, 2), dislist.reshape(B, K)


# speedup vs baseline: 1.1435x; 1.0579x over previous
import jax
import jax.numpy as jnp
from jax.experimental import pallas as pl

B = 64
M = 1000
T = 30
XLEN = 128
K = 6
H = 64
ALPHA = 0.01
THRESH = 0.1
NEG = -1e30


def _lane_mean(x, n):
    s = x * (1.0 / n)
    g = s[:, 0:8]
    for i in range(1, n // 8):
        g = g + s[:, 8 * i:8 * i + 8]
    w = 8
    while w > 1:
        w //= 2
        g = g[:, :w] + g[:, w:2 * w]
    return jax.lax.broadcast_in_dim(g, x.shape, (0, 1))


def _ln(x, g, b):
    mu = _lane_mean(x, x.shape[-1])
    var = _lane_mean((x - mu) ** 2, x.shape[-1])
    return (x - mu) / jnp.sqrt(var + 1e-5) * g + b


def _erfc(x):
    f = jnp.float32
    ax = jnp.abs(x)
    x2 = x * x
    p = x2 * f(7.85386146e-05) + f(-0.000801019371)
    p = p * x2 + f(0.00518832775)
    p = p * x2 + f(-0.0268538129)
    p = p * x2 + f(0.112835854)
    p = p * x2 + f(-0.37612626)
    p = p * x2 + f(1.12837911)
    one_minus_erf = f(1.0) - x * p

    z = -x2
    ez = jnp.exp(z)
    q = ez * (f(1.0) / ax)
    w = f(1.0) / x2
    a = w * f(0.0232682) + f(-0.138703942)
    a = a * w + f(0.368742466)
    a = a * w + f(-0.582473278)
    a = a * w + f(0.621000469)
    a = a * w + f(-0.494451523)
    a = a * w + f(0.340488)
    a = a * w + f(-0.274112701)
    a = a * w + f(0.563825965)
    bb = w * f(-10.477664) + f(12.9772)
    bb = bb * w + f(-7.49551868)
    bb = bb * w + f(2.92101908)
    bb = bb * w + f(-1.01526523)
    bb = bb * w + f(0.42184633)
    bb = bb * w + f(-0.282076746)
    bb = bb * w + f(0.564189494)
    y = q * jnp.where(ax < f(2.0), a, bb)
    y = jnp.where(z < f(-88.7228394), f(0.0), y)
    y = jnp.where(x < f(0.0), f(2.0) - y, y)
    return jnp.where(ax < f(1.0), one_minus_erf, y)


def _gelu(x):
    return 0.5 * x * _erfc(-x / jnp.sqrt(2.0).astype(jnp.float32))


def _sortkey(x):
    b = jax.lax.bitcast_convert_type(x, jnp.int32)
    return jnp.where(b < 0, b ^ jnp.int32(0x7FFFFFFF), b)


def _planes(ints):
    out = []
    for i in range(4):
        c = jax.lax.shift_right_logical(ints, jnp.int32(8 * i))
        out.append(jnp.bitwise_and(c, jnp.int32(255)).astype(jnp.float32))
    return out


def _unplanes(planes):
    acc = jnp.zeros(planes[0].shape, jnp.int32)
    for i in range(4):
        c = planes[i].astype(jnp.int32)
        acc = jnp.bitwise_or(acc, jax.lax.shift_left(c, jnp.int32(8 * i)))
    return acc


def _exact_rowmix(onehot, planes):
    mixed = [jnp.dot(onehot, p, preferred_element_type=jnp.float32)
             for p in planes]
    return jax.lax.bitcast_convert_type(_unplanes(mixed), jnp.float32)


def _body(pre_ref, px_ref, py_ref, at_ref, vf_ref, lx_ref, ly_ref,
          w1p_ref, w1a_ref, w1v_ref, b1_ref, g1_ref, be1_ref,
          w2_ref, b2_ref, g2_ref, be2_ref, w3_ref, b3_ref,
          l3_ref, traj_ref, dis_ref):
    b = pl.program_id(0)

    pre = pre_ref[0]
    px = px_ref[0]
    py = py_ref[0]
    vfb = jnp.broadcast_to(vf_ref[0], (M, XLEN))
    atb = jnp.broadcast_to(at_ref[0], (M, XLEN))
    feat = jnp.concatenate([vfb, atb, pre], axis=1)
    w1full = jnp.concatenate([w1v_ref[...], w1a_ref[...], w1p_ref[...]], axis=0)
    h = jnp.dot(feat, w1full, preferred_element_type=jnp.float32) + b1_ref[...]
    h = _gelu(_ln(h, g1_ref[...], be1_ref[...]))
    h = jnp.dot(h, w2_ref[...], preferred_element_type=jnp.float32) + b2_ref[...]
    h = _gelu(_ln(h, g2_ref[...], be2_ref[...]))
    pd = jnp.dot(h, w3_ref[...], preferred_element_type=jnp.float32) + b3_ref[...]
    pd = pd - jnp.max(pd, axis=0, keepdims=True)
    pd = pd - jnp.log(jnp.sum(jnp.exp(pd), axis=0, keepdims=True))

    d2t = (px - lx_ref[0]) ** 2 + (py - ly_ref[0]) ** 2
    z = -jnp.max(d2t, axis=1, keepdims=True) / ALPHA
    z = z - jnp.max(z, axis=0, keepdims=True)
    e = jnp.exp(z)
    dis = e / jnp.sum(e, axis=0, keepdims=True)
    pos = dis > 0
    logdis = jnp.log(jnp.where(pos, dis, 1.0))
    l3c = jnp.sum(jnp.where(pos, dis * (logdis - pd), 0.0),
                  keepdims=True) * (1.0 / B)

    @pl.when(b == 0)
    def _():
        l3_ref[...] = jnp.zeros((1, 1), jnp.float32)
    l3_ref[...] += l3c

    iota_c = jax.lax.broadcasted_iota(jnp.int32, (M, 1), 0)
    rr2 = jax.lax.broadcasted_iota(jnp.int32, (M, M), 0)
    cc2 = jax.lax.broadcasted_iota(jnp.int32, (M, M), 1)
    eye = jnp.where(rr2 == cc2, 1.0, 0.0)
    ltri = jnp.where(cc2 <= rr2, 1.0, 0.0)

    key = _sortkey(pd)
    pl4 = jnp.concatenate(_planes(key), axis=1)
    tr4 = jax.lax.dot_general(pl4, eye, (((0,), (0,)), ((), ())),
                              preferred_element_type=jnp.float32)
    keyrow = _unplanes([tr4[i:i + 1] for i in range(4)])
    g = jnp.where((key > keyrow) | ((key == keyrow) & (rr2 < cc2)), 1.0, 0.0)
    rank_row = jnp.dot(jnp.ones((1, M), jnp.float32), g,
                       preferred_element_type=jnp.float32)
    perm = jnp.where(iota_c.astype(jnp.float32) == rank_row, 1.0, 0.0)

    hcat = jnp.concatenate([pre, px, py], axis=1)
    hp_c = _planes(jax.lax.bitcast_convert_type(hcat, jnp.int32))
    hplanes = [jnp.dot(perm, p, preferred_element_type=jnp.float32)
               for p in hp_c]
    hcat_p = jax.lax.bitcast_convert_type(_unplanes(hplanes), jnp.float32)
    pxp = hcat_p[:, 2 * T:3 * T]
    pyp = hcat_p[:, 3 * T:]

    n2 = pxp[:, 0:1] ** 2 + pyp[:, 0:1] ** 2
    mask5p = jnp.sqrt(n2) > 5.0

    v = pd
    avail = jnp.ones((M, 1), dtype=jnp.bool_)
    accum = jnp.zeros((M, T), dtype=jnp.float32)
    iota_k = jax.lax.broadcasted_iota(jnp.int32, (1, K), 1)
    dvec = jnp.zeros((1, K), jnp.float32)
    for k in range(K):
        if k == 0:
            pick = iota_c == 0
        else:
            eff = jnp.where(avail, v, NEG)
            mx = jnp.max(eff, axis=0, keepdims=True)
            hit = eff == mx
            cs = jnp.dot(ltri, jnp.where(hit, 1.0, 0.0),
                         preferred_element_type=jnp.float32)
            pick = hit & (cs == 1.0)
        pick_f = jnp.where(pick, 1.0, 0.0)
        vsel = jnp.sum(jnp.where(pick, v, 0.0))
        dvec = jnp.where(iota_k == k, jnp.exp(vsel), dvec)
        rcat = _exact_rowmix(
            jax.lax.dot_general(pick_f, eye, (((0,), (0,)), ((), ())),
                                preferred_element_type=jnp.float32),
            hplanes)
        row = rcat[:, :2 * T]
        rowx = rcat[:, 2 * T:3 * T]
        rowy = rcat[:, 3 * T:]
        traj_ref[0, pl.ds(k, 1), :] = row
        avail = avail & jnp.logical_not(pick)
        v = jnp.where(mask5p, -1.0, v)
        d2 = (pxp - rowx) ** 2 + (pyp - rowy) ** 2
        accum = accum + jnp.sqrt(d2)
        m2p = jnp.max(accum, axis=1, keepdims=True) * (1.0 / (k + 1)) < THRESH
        v = jnp.where(m2p, 1.1 * v, v)
    dis_ref[0] = dvec


def kernel(agent_traj, pre, labels, vectornet_feature,
           W1, b1, g1, be1, W2, b2, g2, be2, W3, b3):
    w1v = W1[:XLEN]
    w1a = W1[XLEN:2 * XLEN]
    w1p = W1[2 * XLEN:]
    pre_r = pre.reshape(B, M, T, 2)
    px = pre_r[..., 0]
    py = pre_r[..., 1]
    labx = labels[..., 0].reshape(B, 1, T)
    laby = labels[..., 1].reshape(B, 1, T)

    fixed = lambda *_: (0, 0)
    l3, traj, dislist = pl.pallas_call(
        _body,
        grid=(B,),
        in_specs=[
            pl.BlockSpec((1, M, 2 * T), lambda b: (b, 0, 0)),
            pl.BlockSpec((1, M, T), lambda b: (b, 0, 0)),
            pl.BlockSpec((1, M, T), lambda b: (b, 0, 0)),
            pl.BlockSpec((1, 1, XLEN), lambda b: (b, 0, 0)),
            pl.BlockSpec((1, 1, XLEN), lambda b: (b, 0, 0)),
            pl.BlockSpec((1, 1, T), lambda b: (b, 0, 0)),
            pl.BlockSpec((1, 1, T), lambda b: (b, 0, 0)),
            pl.BlockSpec((2 * T, H), fixed),
            pl.BlockSpec((XLEN, H), fixed),
            pl.BlockSpec((XLEN, H), fixed),
            pl.BlockSpec((1, H), fixed),
            pl.BlockSpec((1, H), fixed),
            pl.BlockSpec((1, H), fixed),
            pl.BlockSpec((H, H), fixed),
            pl.BlockSpec((1, H), fixed),
            pl.BlockSpec((1, H), fixed),
            pl.BlockSpec((1, H), fixed),
            pl.BlockSpec((H, 1), fixed),
            pl.BlockSpec((1, 1), fixed),
        ],
        out_specs=[
            pl.BlockSpec((1, 1), fixed),
            pl.BlockSpec((1, K, 2 * T), lambda b: (b, 0, 0)),
            pl.BlockSpec((1, 1, K), lambda b: (b, 0, 0)),
        ],
        out_shape=[
            jax.ShapeDtypeStruct((1, 1), jnp.float32),
            jax.ShapeDtypeStruct((B, K, 2 * T), jnp.float32),
            jax.ShapeDtypeStruct((B, 1, K), jnp.float32),
        ],
    )(pre, px, py,
      agent_traj.reshape(B, 1, XLEN), vectornet_feature.reshape(B, 1, XLEN),
      labx, laby,
      w1p, w1a, w1v, b1.reshape(1, H), g1.reshape(1, H), be1.reshape(1, H),
      W2, b2.reshape(1, H), g2.reshape(1, H), be2.reshape(1, H),
      W3, b3.reshape(1, 1))
    return l3[0, 0], traj.reshape(B, K, T, 2), dislist.reshape(B, K)


# speedup vs baseline: 1.6845x; 1.4731x over previous
import jax
import jax.numpy as jnp
from jax.experimental import pallas as pl

B = 64
M = 1000
T = 30
XLEN = 128
K = 6
H = 64
ALPHA = 0.01
THRESH = 0.1
NEG = -1e30


def _lane_mean(x, n):
    s = x * (1.0 / n)
    g = s[:, 0:8]
    for i in range(1, n // 8):
        g = g + s[:, 8 * i:8 * i + 8]
    w = 8
    while w > 1:
        w //= 2
        g = g[:, :w] + g[:, w:2 * w]
    return jax.lax.broadcast_in_dim(g, x.shape, (0, 1))


def _ln(x, g, b):
    mu = _lane_mean(x, x.shape[-1])
    var = _lane_mean((x - mu) ** 2, x.shape[-1])
    return (x - mu) / jnp.sqrt(var + 1e-5) * g + b


def _erfc(x):
    f = jnp.float32
    ax = jnp.abs(x)
    x2 = x * x
    p = x2 * f(7.85386146e-05) + f(-0.000801019371)
    p = p * x2 + f(0.00518832775)
    p = p * x2 + f(-0.0268538129)
    p = p * x2 + f(0.112835854)
    p = p * x2 + f(-0.37612626)
    p = p * x2 + f(1.12837911)
    one_minus_erf = f(1.0) - x * p

    z = -x2
    ez = jnp.exp(z)
    q = ez * (f(1.0) / ax)
    w = f(1.0) / x2
    a = w * f(0.0232682) + f(-0.138703942)
    a = a * w + f(0.368742466)
    a = a * w + f(-0.582473278)
    a = a * w + f(0.621000469)
    a = a * w + f(-0.494451523)
    a = a * w + f(0.340488)
    a = a * w + f(-0.274112701)
    a = a * w + f(0.563825965)
    bb = w * f(-10.477664) + f(12.9772)
    bb = bb * w + f(-7.49551868)
    bb = bb * w + f(2.92101908)
    bb = bb * w + f(-1.01526523)
    bb = bb * w + f(0.42184633)
    bb = bb * w + f(-0.282076746)
    bb = bb * w + f(0.564189494)
    y = q * jnp.where(ax < f(2.0), a, bb)
    y = jnp.where(z < f(-88.7228394), f(0.0), y)
    y = jnp.where(x < f(0.0), f(2.0) - y, y)
    return jnp.where(ax < f(1.0), one_minus_erf, y)


def _gelu(x):
    return 0.5 * x * _erfc(-x / jnp.sqrt(2.0).astype(jnp.float32))


def _sortkey(x):
    b = jax.lax.bitcast_convert_type(x, jnp.int32)
    return jnp.where(b < 0, b ^ jnp.int32(0x7FFFFFFF), b)


def _planes(ints):
    out = []
    for i in range(4):
        c = jax.lax.shift_right_logical(ints, jnp.int32(8 * i))
        out.append(jnp.bitwise_and(c, jnp.int32(255)).astype(jnp.float32))
    return out


def _unplanes(planes):
    acc = jnp.zeros(planes[0].shape, jnp.int32)
    for i in range(4):
        c = planes[i].astype(jnp.int32)
        acc = jnp.bitwise_or(acc, jax.lax.shift_left(c, jnp.int32(8 * i)))
    return acc


def _exact_rowmix(onehot, planes):
    mixed = [jnp.dot(onehot, p, preferred_element_type=jnp.float32)
             for p in planes]
    return jax.lax.bitcast_convert_type(_unplanes(mixed), jnp.float32)


def _body(pre_ref, px_ref, py_ref, at_ref, vf_ref, lx_ref, ly_ref,
          w1p_ref, w1a_ref, w1v_ref, b1_ref, g1_ref, be1_ref,
          w2_ref, b2_ref, g2_ref, be2_ref, w3_ref, b3_ref,
          l3_ref, traj_ref, dis_ref):
    b = pl.program_id(0)

    pre = pre_ref[0]
    px = px_ref[0]
    py = py_ref[0]
    vfb = jnp.broadcast_to(vf_ref[0], (M, XLEN))
    atb = jnp.broadcast_to(at_ref[0], (M, XLEN))
    feat = jnp.concatenate([vfb, atb, pre], axis=1)
    w1full = jnp.concatenate([w1v_ref[...], w1a_ref[...], w1p_ref[...]], axis=0)
    h = jnp.dot(feat, w1full, preferred_element_type=jnp.float32) + b1_ref[...]
    h = _gelu(_ln(h, g1_ref[...], be1_ref[...]))
    h = jnp.dot(h, w2_ref[...], preferred_element_type=jnp.float32) + b2_ref[...]
    h = _gelu(_ln(h, g2_ref[...], be2_ref[...]))
    pd = jnp.dot(h, w3_ref[...], preferred_element_type=jnp.float32) + b3_ref[...]
    pd = pd - jnp.max(pd, axis=0, keepdims=True)
    pd = pd - jnp.log(jnp.sum(jnp.exp(pd), axis=0, keepdims=True))

    d2t = (px - lx_ref[0]) ** 2 + (py - ly_ref[0]) ** 2
    z = -jnp.max(d2t, axis=1, keepdims=True) / ALPHA
    z = z - jnp.max(z, axis=0, keepdims=True)
    e = jnp.exp(z)
    dis = e / jnp.sum(e, axis=0, keepdims=True)
    pos = dis > 0
    logdis = jnp.log(jnp.where(pos, dis, 1.0))
    l3c = jnp.sum(jnp.where(pos, dis * (logdis - pd), 0.0),
                  keepdims=True) * (1.0 / B)

    @pl.when(b == 0)
    def _():
        l3_ref[...] = jnp.zeros((1, 1), jnp.float32)
    l3_ref[...] += l3c

    iota_c = jax.lax.broadcasted_iota(jnp.int32, (M, 1), 0)
    rr2 = jax.lax.broadcasted_iota(jnp.int32, (M, M), 0)
    cc2 = jax.lax.broadcasted_iota(jnp.int32, (M, M), 1)
    eye = jnp.where(rr2 == cc2, 1.0, 0.0)

    key = _sortkey(pd)
    pl4 = jnp.concatenate(_planes(key), axis=1)
    tr4 = jax.lax.dot_general(pl4, eye, (((0,), (0,)), ((), ())),
                              preferred_element_type=jnp.float32)
    keyrow = _unplanes([tr4[i:i + 1] for i in range(4)])
    g = jnp.where((key > keyrow) | ((key == keyrow) & (rr2 < cc2)), 1.0, 0.0)
    rank_row = jnp.dot(jnp.ones((1, M), jnp.float32), g,
                       preferred_element_type=jnp.float32)
    perm = jnp.where(iota_c.astype(jnp.float32) == rank_row, 1.0, 0.0)

    hcat = jnp.concatenate([pre, px, py], axis=1)
    hp_c = _planes(jax.lax.bitcast_convert_type(hcat, jnp.int32))
    hplanes = [jnp.dot(perm, p, preferred_element_type=jnp.float32)
               for p in hp_c]
    hcat_p = jax.lax.bitcast_convert_type(_unplanes(hplanes), jnp.float32)
    pxp = hcat_p[:, 2 * T:3 * T]
    pyp = hcat_p[:, 3 * T:]

    n2 = pxp[:, 0:1] ** 2 + pyp[:, 0:1] ** 2
    mask5p = jnp.sqrt(n2) > 5.0

    v = pd
    avail = jnp.ones((M, 1), dtype=jnp.bool_)
    accum = jnp.zeros((M, T), dtype=jnp.float32)
    iota_k = jax.lax.broadcasted_iota(jnp.int32, (1, K), 1)
    iota_r = jax.lax.broadcasted_iota(jnp.int32, (1, M), 1)
    dvec = jnp.zeros((1, K), jnp.float32)
    for k in range(K):
        if k == 0:
            idxm = jnp.zeros((1, 1), jnp.int32)
        else:
            eff = jnp.where(avail, v, NEG)
            mx = jnp.max(eff, axis=0, keepdims=True)
            idxm = jnp.min(jnp.where(eff == mx, iota_c, M),
                           axis=0, keepdims=True)
        pick = iota_c == idxm
        vsel = jnp.sum(jnp.where(pick, v, 0.0))
        dvec = jnp.where(iota_k == k, jnp.exp(vsel), dvec)
        rcat = _exact_rowmix(
            jnp.where(iota_r == idxm, 1.0, 0.0),
            hplanes)
        row = rcat[:, :2 * T]
        rowx = rcat[:, 2 * T:3 * T]
        rowy = rcat[:, 3 * T:]
        traj_ref[0, pl.ds(k, 1), :] = row
        avail = avail & jnp.logical_not(pick)
        v = jnp.where(mask5p, -1.0, v)
        d2 = (pxp - rowx) ** 2 + (pyp - rowy) ** 2
        accum = accum + jnp.sqrt(d2)
        m2p = jnp.max(accum, axis=1, keepdims=True) * (1.0 / (k + 1)) < THRESH
        v = jnp.where(m2p, 1.1 * v, v)
    dis_ref[0] = dvec


def kernel(agent_traj, pre, labels, vectornet_feature,
           W1, b1, g1, be1, W2, b2, g2, be2, W3, b3):
    w1v = W1[:XLEN]
    w1a = W1[XLEN:2 * XLEN]
    w1p = W1[2 * XLEN:]
    pre_r = pre.reshape(B, M, T, 2)
    px = pre_r[..., 0]
    py = pre_r[..., 1]
    labx = labels[..., 0].reshape(B, 1, T)
    laby = labels[..., 1].reshape(B, 1, T)

    fixed = lambda *_: (0, 0)
    l3, traj, dislist = pl.pallas_call(
        _body,
        grid=(B,),
        in_specs=[
            pl.BlockSpec((1, M, 2 * T), lambda b: (b, 0, 0)),
            pl.BlockSpec((1, M, T), lambda b: (b, 0, 0)),
            pl.BlockSpec((1, M, T), lambda b: (b, 0, 0)),
            pl.BlockSpec((1, 1, XLEN), lambda b: (b, 0, 0)),
            pl.BlockSpec((1, 1, XLEN), lambda b: (b, 0, 0)),
            pl.BlockSpec((1, 1, T), lambda b: (b, 0, 0)),
            pl.BlockSpec((1, 1, T), lambda b: (b, 0, 0)),
            pl.BlockSpec((2 * T, H), fixed),
            pl.BlockSpec((XLEN, H), fixed),
            pl.BlockSpec((XLEN, H), fixed),
            pl.BlockSpec((1, H), fixed),
            pl.BlockSpec((1, H), fixed),
            pl.BlockSpec((1, H), fixed),
            pl.BlockSpec((H, H), fixed),
            pl.BlockSpec((1, H), fixed),
            pl.BlockSpec((1, H), fixed),
            pl.BlockSpec((1, H), fixed),
            pl.BlockSpec((H, 1), fixed),
            pl.BlockSpec((1, 1), fixed),
        ],
        out_specs=[
            pl.BlockSpec((1, 1), fixed),
            pl.BlockSpec((1, K, 2 * T), lambda b: (b, 0, 0)),
            pl.BlockSpec((1, 1, K), lambda b: (b, 0, 0)),
        ],
        out_shape=[
            jax.ShapeDtypeStruct((1, 1), jnp.float32),
            jax.ShapeDtypeStruct((B, K, 2 * T), jnp.float32),
            jax.ShapeDtypeStruct((B, 1, K), jnp.float32),
        ],
    )(pre, px, py,
      agent_traj.reshape(B, 1, XLEN), vectornet_feature.reshape(B, 1, XLEN),
      labx, laby,
      w1p, w1a, w1v, b1.reshape(1, H), g1.reshape(1, H), be1.reshape(1, H),
      W2, b2.reshape(1, H), g2.reshape(1, H), be2.reshape(1, H),
      W3, b3.reshape(1, 1))
    return l3[0, 0], traj.reshape(B, K, T, 2), dislist.reshape(B, K)
